# baseline (speedup 1.0000x reference)
import functools

import jax
import jax.numpy as jnp
from jax.experimental import pallas as pl
from jax.experimental.pallas import tpu as pltpu


_EPS = 1e-5


def _split_hi_lo(w):
    hi = w.astype(jnp.bfloat16)
    lo = (w - hi.astype(jnp.float32)).astype(jnp.bfloat16)
    return hi, lo


def _body(T, B, HW, C, HID, E,
          x_ref, rwh_ref, rwl_ref, rg_ref, rbb_ref,
          w1h_ref, w1l_ref, wc2_ref,
          g1_ref, bb1_ref, g2p_ref, bb2p_ref, g2c_ref, bb2c_ref,
          tau_ref, o_ref, wgt_ref, y2b_ref):
    p = pl.program_id(0)
    N = T * B
    ROWS = N * HW
    xs = x_ref[...]
    iota = jax.lax.broadcasted_iota(jnp.int32, (N, E), 1)

    @pl.when(p == 0)
    def _router():
        xr = xs.reshape(T, B * HW, C)
        v = jnp.zeros_like(xr[0])
        spikes = []
        for t in range(T):
            v = v + (xr[t] - v) / 2.0
            s = (v - 1.0 >= 0.0).astype(jnp.float32)
            spikes.append(s)
            v = v * (1.0 - s)
        S = jnp.concatenate(spikes, axis=0).astype(jnp.bfloat16)
        y = (jax.lax.dot_general(S, rwh_ref[...], (((1,), (0,)), ((), ())),
                                 preferred_element_type=jnp.float32)
             + jax.lax.dot_general(S, rwl_ref[...], (((1,), (0,)), ((), ())),
                                   preferred_element_type=jnp.float32))
        m = jnp.mean(y, axis=0, keepdims=True)
        var = jnp.mean(y * y, axis=0, keepdims=True) - m * m
        ybn = (y - m) * (rg_ref[...] * jax.lax.rsqrt(var + _EPS)) + rbb_ref[...]
        logits = jnp.mean(ybn.reshape(N, HW, E), axis=1)
        mx = jnp.max(logits, axis=1, keepdims=True)
        ex = jnp.exp(logits - mx)
        prob = ex / jnp.sum(ex, axis=1, keepdims=True)
        p1 = jnp.max(prob, axis=1, keepdims=True)
        i1 = jnp.min(jnp.where(prob == p1, iota, E), axis=1, keepdims=True)
        pm = jnp.where(iota == i1, -jnp.inf, prob)
        p2 = jnp.max(pm, axis=1, keepdims=True)
        i2 = jnp.min(jnp.where(pm == p2, iota, E), axis=1, keepdims=True)
        tot = p1 + p2
        wgt_ref[...] = (jnp.where(iota == i1, p1 / tot, 0.0)
                        + jnp.where(iota == i2, p2 / tot, 0.0))

    def _finish(expert, g2_blk, bb2_blk):
        y2 = y2b_ref[:, :C] + y2b_ref[:, C:]
        m2 = jnp.mean(y2, axis=0, keepdims=True)
        v2 = jnp.mean(y2 * y2, axis=0, keepdims=True) - m2 * m2
        sc2 = g2_blk * jax.lax.rsqrt(v2 + _EPS)
        y2n = (y2 - m2) * sc2 + bb2_blk
        we = jnp.sum(wgt_ref[...] * (iota == expert).astype(jnp.float32),
                     axis=1)
        return (y2n.reshape(N, HW, C) * we[:, None, None]).reshape(ROWS, C)

    contrib = _finish(p - 1, g2p_ref[0], bb2p_ref[0])
    o_ref[...] = jnp.where(p == 0, xs, o_ref[...] + contrib)

    tau = tau_ref[0, 0, 0]
    s1 = (xs >= tau).astype(jnp.bfloat16)
    y1 = (jax.lax.dot_general(s1, w1h_ref[0], (((1,), (0,)), ((), ())),
                              preferred_element_type=jnp.float32)
          + jax.lax.dot_general(s1, w1l_ref[0], (((1,), (0,)), ((), ())),
                                preferred_element_type=jnp.float32))
    m1 = jnp.mean(y1, axis=0, keepdims=True)
    v1 = jnp.mean(y1 * y1, axis=0, keepdims=True) - m1 * m1
    sc1 = g1_ref[0] * jax.lax.rsqrt(v1 + _EPS)
    thr1 = m1 + (tau - bb1_ref[0]) / sc1
    s2 = (y1 >= thr1).astype(jnp.bfloat16)
    y2b_ref[...] = jax.lax.dot_general(s2, wc2_ref[0],
                                       (((1,), (0,)), ((), ())),
                                       preferred_element_type=jnp.float32)

    @pl.when(p == E - 1)
    def _last():
        o_ref[...] = o_ref[...] + _finish(E - 1, g2c_ref[0], bb2c_ref[0])


def kernel(x, router_w, router_b, router_bn_g, router_bn_b,
           w1, b1, bn1_g, bn1_b, w2, b2, bn2_g, bn2_b):
    T, B, C, H, W = x.shape
    HW = H * W
    N = T * B
    ROWS = N * HW
    E, HID, _ = w1.shape
    taus = [1.9 + i * (2.1 - 1.9) / (E - 1) for i in range(E)]

    xm = jnp.transpose(x.reshape(T, B, C, HW), (0, 1, 3, 2)).reshape(ROWS, C)

    rwh, rwl = _split_hi_lo(router_w.T)
    w1h, w1l = _split_hi_lo(jnp.transpose(w1, (0, 2, 1)))
    w2h, w2l = _split_hi_lo(jnp.transpose(w2, (0, 2, 1)))
    w2c = jnp.concatenate([w2h, w2l], axis=2)
    del router_b, b1, b2

    g2r = bn2_g.reshape(E, 1, C)
    bb2r = bn2_b.reshape(E, 1, C)
    prev = lambda e: (jnp.maximum(e - 1, 0), 0, 0)
    cur = lambda e: (e, 0, 0)
    tau_arr = jnp.asarray(taus, dtype=jnp.float32).reshape(E, 1, 1)
    fused = pl.pallas_call(
        functools.partial(_body, T, B, HW, C, HID, E),
        grid=(E,),
        in_specs=[
            pl.BlockSpec((ROWS, C), lambda e: (0, 0)),
            pl.BlockSpec((C, E), lambda e: (0, 0)),
            pl.BlockSpec((C, E), lambda e: (0, 0)),
            pl.BlockSpec((1, E), lambda e: (0, 0)),
            pl.BlockSpec((1, E), lambda e: (0, 0)),
            pl.BlockSpec((1, C, HID), cur),
            pl.BlockSpec((1, C, HID), cur),
            pl.BlockSpec((1, HID, 2 * C), cur),
            pl.BlockSpec((1, 1, HID), cur),
            pl.BlockSpec((1, 1, HID), cur),
            pl.BlockSpec((1, 1, C), prev),
            pl.BlockSpec((1, 1, C), prev),
            pl.BlockSpec((1, 1, C), cur),
            pl.BlockSpec((1, 1, C), cur),
            pl.BlockSpec((1, 1, 1), cur),
        ],
        out_specs=pl.BlockSpec((ROWS, C), lambda e: (0, 0)),
        out_shape=jax.ShapeDtypeStruct((ROWS, C), jnp.float32),
        scratch_shapes=[pltpu.VMEM((N, E), jnp.float32),
                        pltpu.VMEM((ROWS, 2 * C), jnp.float32)],
    )
    out = fused(xm, rwh, rwl,
                router_bn_g.reshape(1, E), router_bn_b.reshape(1, E),
                w1h, w1l, w2c,
                bn1_g.reshape(E, 1, HID), bn1_b.reshape(E, 1, HID),
                g2r, bb2r, g2r, bb2r,
                tau_arr)

    return jnp.transpose(out.reshape(T, B, HW, C), (0, 1, 3, 2)).reshape(
        T, B, C, H, W)


# speedup vs baseline: 1.0441x; 1.0441x over previous
import functools

import jax
import jax.numpy as jnp
from jax.experimental import pallas as pl
from jax.experimental.pallas import tpu as pltpu


_EPS = 1e-5


def _split_hi_lo(w):
    hi = w.astype(jnp.bfloat16)
    lo = (w - hi.astype(jnp.float32)).astype(jnp.bfloat16)
    return hi, lo


def _body(T, B, HW, C, HID, E, EC,
          x_ref, rwh_ref, rwl_ref, rg_ref, rbb_ref,
          w1h_ref, w1l_ref, wc2_ref,
          g1_ref, bb1_ref, g2_ref, bb2_ref,
          tau_ref, o_ref, wgt_ref):
    c = pl.program_id(0)
    j = pl.program_id(1)
    e = c * EC + j
    N = T * B
    ROWS = N * HW
    xs = x_ref[...]

    @pl.when(j == 0)
    def _router():
        xr = xs.reshape(T, B * HW, C)
        v = jnp.zeros_like(xr[0])
        spikes = []
        for t in range(T):
            v = v + (xr[t] - v) / 2.0
            s = (v - 1.0 >= 0.0).astype(jnp.float32)
            spikes.append(s)
            v = v * (1.0 - s)
        S = jnp.concatenate(spikes, axis=0).astype(jnp.bfloat16)
        y = (jax.lax.dot_general(S, rwh_ref[...], (((1,), (0,)), ((), ())),
                                 preferred_element_type=jnp.float32)
             + jax.lax.dot_general(S, rwl_ref[...], (((1,), (0,)), ((), ())),
                                   preferred_element_type=jnp.float32))
        m = jnp.mean(y, axis=0, keepdims=True)
        var = jnp.mean(y * y, axis=0, keepdims=True) - m * m
        ybn = (y - m) * (rg_ref[...] * jax.lax.rsqrt(var + _EPS)) + rbb_ref[...]
        logits = jnp.mean(ybn.reshape(N, HW, E), axis=1)
        mx = jnp.max(logits, axis=1, keepdims=True)
        ex = jnp.exp(logits - mx)
        p = ex / jnp.sum(ex, axis=1, keepdims=True)
        iota = jax.lax.broadcasted_iota(jnp.int32, (N, E), 1)
        m1 = jnp.max(p, axis=1, keepdims=True)
        i1 = jnp.min(jnp.where(p == m1, iota, E), axis=1, keepdims=True)
        p2 = jnp.where(iota == i1, -jnp.inf, p)
        m2 = jnp.max(p2, axis=1, keepdims=True)
        i2 = jnp.min(jnp.where(p2 == m2, iota, E), axis=1, keepdims=True)
        tot = m1 + m2
        wgt_ref[...] = (jnp.where(iota == i1, m1 / tot, 0.0)
                        + jnp.where(iota == i2, m2 / tot, 0.0))

    tau = tau_ref[0, 0, 0]
    s1 = (xs >= tau).astype(jnp.bfloat16)
    y1 = (jax.lax.dot_general(s1, w1h_ref[0], (((1,), (0,)), ((), ())),
                              preferred_element_type=jnp.float32)
          + jax.lax.dot_general(s1, w1l_ref[0], (((1,), (0,)), ((), ())),
                                preferred_element_type=jnp.float32))
    m1 = jnp.mean(y1, axis=0, keepdims=True)
    v1 = jnp.mean(y1 * y1, axis=0, keepdims=True) - m1 * m1
    sc1 = g1_ref[0] * jax.lax.rsqrt(v1 + _EPS)
    thr1 = m1 + (tau - bb1_ref[0]) / sc1
    s2 = (y1 >= thr1).astype(jnp.bfloat16)
    y2b = jax.lax.dot_general(s2, wc2_ref[0], (((1,), (0,)), ((), ())),
                              preferred_element_type=jnp.float32)
    y2 = y2b[:, :C] + y2b[:, C:]
    m2 = jnp.mean(y2, axis=0, keepdims=True)
    v2 = jnp.mean(y2 * y2, axis=0, keepdims=True) - m2 * m2
    sc2 = g2_ref[0] * jax.lax.rsqrt(v2 + _EPS)
    y2n = (y2 - m2) * sc2 + bb2_ref[0]
    iota = jax.lax.broadcasted_iota(jnp.int32, (N, E), 1)
    we = jnp.sum(wgt_ref[...] * (iota == e).astype(jnp.float32), axis=1)
    contrib = (y2n.reshape(N, HW, C) * we[:, None, None]).reshape(ROWS, C)

    @pl.when(j == 0)
    def _():
        o_ref[0] = contrib

    @pl.when(j != 0)
    def _():
        o_ref[0] = o_ref[0] + contrib


def kernel(x, router_w, router_b, router_bn_g, router_bn_b,
           w1, b1, bn1_g, bn1_b, w2, b2, bn2_g, bn2_b):
    T, B, C, H, W = x.shape
    HW = H * W
    N = T * B
    ROWS = N * HW
    E, HID, _ = w1.shape
    EC = E // 2
    taus = [1.9 + i * (2.1 - 1.9) / (E - 1) for i in range(E)]

    xm = jnp.transpose(x.reshape(T, B, C, HW), (0, 1, 3, 2)).reshape(ROWS, C)

    rwh, rwl = _split_hi_lo(router_w.T)
    w1h, w1l = _split_hi_lo(jnp.transpose(w1, (0, 2, 1)))
    w2h, w2l = _split_hi_lo(jnp.transpose(w2, (0, 2, 1)))
    w2c = jnp.concatenate([w2h, w2l], axis=2)
    del router_b, b1, b2

    exp3 = lambda c, j: (c * EC + j, 0, 0)
    full = lambda c, j: (0, 0)
    tau_arr = jnp.asarray(taus, dtype=jnp.float32).reshape(E, 1, 1)
    fused = pl.pallas_call(
        functools.partial(_body, T, B, HW, C, HID, E, EC),
        grid=(2, EC),
        in_specs=[
            pl.BlockSpec((ROWS, C), full),
            pl.BlockSpec((C, E), full),
            pl.BlockSpec((C, E), full),
            pl.BlockSpec((1, E), full),
            pl.BlockSpec((1, E), full),
            pl.BlockSpec((1, C, HID), exp3),
            pl.BlockSpec((1, C, HID), exp3),
            pl.BlockSpec((1, HID, 2 * C), exp3),
            pl.BlockSpec((1, 1, HID), exp3),
            pl.BlockSpec((1, 1, HID), exp3),
            pl.BlockSpec((1, 1, C), exp3),
            pl.BlockSpec((1, 1, C), exp3),
            pl.BlockSpec((1, 1, 1), exp3),
        ],
        out_specs=pl.BlockSpec((1, ROWS, C), lambda c, j: (c, 0, 0)),
        out_shape=jax.ShapeDtypeStruct((2, ROWS, C), jnp.float32),
        scratch_shapes=[pltpu.VMEM((N, E), jnp.float32)],
        compiler_params=pltpu.CompilerParams(
            dimension_semantics=("parallel", "arbitrary")),
    )
    parts = fused(xm, rwh, rwl,
                  router_bn_g.reshape(1, E), router_bn_b.reshape(1, E),
                  w1h, w1l, w2c,
                  bn1_g.reshape(E, 1, HID), bn1_b.reshape(E, 1, HID),
                  bn2_g.reshape(E, 1, C), bn2_b.reshape(E, 1, C),
                  tau_arr)

    out = parts[0] + parts[1] + xm
    return jnp.transpose(out.reshape(T, B, HW, C), (0, 1, 3, 2)).reshape(
        T, B, C, H, W)
